# baseline (speedup 1.0000x reference)
import functools

import jax
import jax.numpy as jnp
from jax import lax
from jax.experimental import pallas as pl
from jax.experimental.pallas import tpu as pltpu
from jax.experimental.pallas import tpu_sc as plsc

B = 16384
D = 3072
N_AC = 36
AC_SLOT = 23
AC_BASE = 8196
AC_ROWS = N_AC * AC_SLOT
AC_PAD = 832

_NC = 2
_NS = 16
_NW = _NC * _NS
_CH = 16
_NSLICE = 4
_BS = B // _NSLICE
_B_PER_W = _BS // _NW
_NCH = _B_PER_W // _CH


def _sc_gather(idx3, table):
    mesh = plsc.VectorSubcoreMesh(core_axis_name="c", subcore_axis_name="s")

    @functools.partial(
        pl.kernel,
        mesh=mesh,
        out_type=jax.ShapeDtypeStruct((_BS, D), jnp.float32),
        scratch_types=[
            pltpu.VMEM((_NCH, _CH), jnp.int32),
            pltpu.VMEM((2, _CH, D), jnp.float32),
            pltpu.SemaphoreType.DMA,
            pltpu.SemaphoreType.DMA,
        ],
    )
    def k(idx_hbm, table_hbm, out_hbm, idx_v, buf_v, gsem, osem):
        wid = lax.axis_index("s") * _NC + lax.axis_index("c")
        base = wid * _B_PER_W
        pltpu.sync_copy(idx_hbm.at[wid], idx_v)
        pltpu.async_copy(table_hbm.at[idx_v.at[0]], buf_v.at[0], gsem)

        def body(g, carry):
            slot = lax.rem(g, 2)
            nxt = 1 - slot

            @pl.when(g >= 1)
            def _():
                pltpu.make_async_copy(
                    buf_v.at[nxt], out_hbm.at[pl.ds(base + (g - 1) * _CH, _CH)], osem
                ).wait()

            @pl.when(g + 1 < _NCH)
            def _():
                pltpu.async_copy(table_hbm.at[idx_v.at[g + 1]], buf_v.at[nxt], gsem)

            pltpu.make_async_copy(table_hbm.at[idx_v.at[g]], buf_v.at[slot], gsem).wait()
            pltpu.async_copy(buf_v.at[slot], out_hbm.at[pl.ds(base + g * _CH, _CH)], osem)
            return carry

        lax.fori_loop(0, _NCH, body, 0)
        pltpu.make_async_copy(
            buf_v.at[(_NCH - 1) % 2],
            out_hbm.at[pl.ds(base + (_NCH - 1) * _CH, _CH)],
            osem,
        ).wait()

    return k(idx3, table)


_TB = 256


def _combine_body(codes_ref, s_ref, tac_ref, o_ref):
    codes = codes_ref[...].astype(jnp.float32)
    s = s_ref[...].astype(jnp.float32)
    i_of = lax.broadcasted_iota(jnp.int32, (N_AC, AC_PAD), 0)
    j_of = lax.broadcasted_iota(jnp.int32, (N_AC, AC_PAD), 1)
    rep_mat = (i_of == j_of // AC_SLOT).astype(jnp.float32)
    rep = jnp.dot(codes, rep_mat, preferred_element_type=jnp.float32)
    m = (lax.broadcasted_iota(jnp.int32, (_TB, AC_PAD), 1) % AC_SLOT)
    oh = (rep == m.astype(jnp.float32)).astype(jnp.bfloat16)
    ac = jnp.dot(oh, tac_ref[...], preferred_element_type=jnp.float32)
    o_ref[...] = (s + ac)[:, None, :]


def _combine_body_chained(codes_ref, s_ref, tac_ref, prev_ref, o_ref):
    del prev_ref
    _combine_body(codes_ref, s_ref, tac_ref, o_ref)


def _tc_combine(codes, s, tac, blk0, prev=None):
    grid = (_BS // _TB,)
    in_specs = [
        pl.BlockSpec((_TB, N_AC), lambda i: (i, 0)),
        pl.BlockSpec((_TB, D), lambda i: (i, 0)),
        pl.BlockSpec((AC_PAD, D), lambda i: (0, 0)),
    ]
    args = [codes, s, tac]
    body = _combine_body
    aliases = {}
    if prev is not None:
        in_specs.append(pl.BlockSpec(memory_space=pl.ANY))
        args.append(prev)
        body = _combine_body_chained
        aliases = {3: 0}
    return pl.pallas_call(
        body,
        grid=grid,
        in_specs=in_specs,
        out_specs=pl.BlockSpec((_TB, 1, D), lambda i: (i + blk0, 0, 0)),
        out_shape=jax.ShapeDtypeStruct((B, 1, D), jnp.float32),
        input_output_aliases=aliases,
        compiler_params=pltpu.CompilerParams(
            dimension_semantics=("arbitrary",),
        ),
    )(*args)


def kernel(semantic_code, acoustic_codes, table):
    sem_idx = semantic_code.reshape(B).astype(jnp.int32) + 2
    codes = acoustic_codes.astype(jnp.int32)
    tac = jnp.concatenate(
        [
            table[AC_BASE : AC_BASE + AC_ROWS],
            jnp.zeros((AC_PAD - AC_ROWS, D), jnp.float32),
        ]
    ).astype(jnp.bfloat16)
    s_slices = [
        _sc_gather(
            sem_idx[k * _BS : (k + 1) * _BS].reshape(_NW, _NCH, _CH), table
        )
        for k in range(_NSLICE)
    ]
    out = None
    for k in range(_NSLICE):
        out = _tc_combine(
            codes[k * _BS : (k + 1) * _BS],
            s_slices[k],
            tac,
            k * (_BS // _TB),
            prev=out,
        )
    return out


# speedup vs baseline: 1.0050x; 1.0050x over previous
import functools

import jax
import jax.numpy as jnp
from jax import lax
from jax.experimental import pallas as pl
from jax.experimental.pallas import tpu as pltpu
from jax.experimental.pallas import tpu_sc as plsc

B = 16384
D = 3072
N_AC = 36
AC_SLOT = 23
AC_BASE = 8196
AC_ROWS = N_AC * AC_SLOT
AC_PAD = 832

_NC = 2
_NS = 16
_NW = _NC * _NS
_CH = 16
_NSLICE = 2
_BS = B // _NSLICE
_B_PER_W = _BS // _NW
_NCH = _B_PER_W // _CH


def _sc_gather(idx3, table):
    mesh = plsc.VectorSubcoreMesh(core_axis_name="c", subcore_axis_name="s")

    @functools.partial(
        pl.kernel,
        mesh=mesh,
        out_type=jax.ShapeDtypeStruct((_BS, D), jnp.float32),
        scratch_types=[
            pltpu.VMEM((_NCH, _CH), jnp.int32),
            pltpu.VMEM((2, _CH, D), jnp.float32),
            pltpu.SemaphoreType.DMA,
            pltpu.SemaphoreType.DMA,
        ],
    )
    def k(idx_hbm, table_hbm, out_hbm, idx_v, buf_v, gsem, osem):
        wid = lax.axis_index("s") * _NC + lax.axis_index("c")
        base = wid * _B_PER_W
        pltpu.sync_copy(idx_hbm.at[wid], idx_v)
        pltpu.async_copy(table_hbm.at[idx_v.at[0]], buf_v.at[0], gsem)

        def body(g, carry):
            slot = lax.rem(g, 2)
            nxt = 1 - slot

            @pl.when(g >= 1)
            def _():
                pltpu.make_async_copy(
                    buf_v.at[nxt], out_hbm.at[pl.ds(base + (g - 1) * _CH, _CH)], osem
                ).wait()

            @pl.when(g + 1 < _NCH)
            def _():
                pltpu.async_copy(table_hbm.at[idx_v.at[g + 1]], buf_v.at[nxt], gsem)

            pltpu.make_async_copy(table_hbm.at[idx_v.at[g]], buf_v.at[slot], gsem).wait()
            pltpu.async_copy(buf_v.at[slot], out_hbm.at[pl.ds(base + g * _CH, _CH)], osem)
            return carry

        lax.fori_loop(0, _NCH, body, 0)
        pltpu.make_async_copy(
            buf_v.at[(_NCH - 1) % 2],
            out_hbm.at[pl.ds(base + (_NCH - 1) * _CH, _CH)],
            osem,
        ).wait()

    return k(idx3, table)


_TB = 256


def _combine_body(codes_ref, s_ref, tac_ref, o_ref):
    codes = codes_ref[...].astype(jnp.float32)
    s = s_ref[...].astype(jnp.float32)
    i_of = lax.broadcasted_iota(jnp.int32, (N_AC, AC_PAD), 0)
    j_of = lax.broadcasted_iota(jnp.int32, (N_AC, AC_PAD), 1)
    rep_mat = (i_of == j_of // AC_SLOT).astype(jnp.float32)
    rep = jnp.dot(codes, rep_mat, preferred_element_type=jnp.float32)
    m = (lax.broadcasted_iota(jnp.int32, (_TB, AC_PAD), 1) % AC_SLOT)
    oh = (rep == m.astype(jnp.float32)).astype(jnp.bfloat16)
    ac = jnp.dot(oh, tac_ref[...], preferred_element_type=jnp.float32)
    o_ref[...] = (s + ac)[:, None, :]


def _combine_body_chained(codes_ref, s_ref, tac_ref, prev_ref, o_ref):
    del prev_ref
    _combine_body(codes_ref, s_ref, tac_ref, o_ref)


def _tc_combine(codes, s, tac, blk0, prev=None):
    grid = (_BS // _TB,)
    in_specs = [
        pl.BlockSpec((_TB, N_AC), lambda i: (i, 0)),
        pl.BlockSpec((_TB, D), lambda i: (i, 0)),
        pl.BlockSpec((AC_PAD, D), lambda i: (0, 0)),
    ]
    args = [codes, s, tac]
    body = _combine_body
    aliases = {}
    if prev is not None:
        in_specs.append(pl.BlockSpec(memory_space=pl.ANY))
        args.append(prev)
        body = _combine_body_chained
        aliases = {3: 0}
    return pl.pallas_call(
        body,
        grid=grid,
        in_specs=in_specs,
        out_specs=pl.BlockSpec((_TB, 1, D), lambda i: (i + blk0, 0, 0)),
        out_shape=jax.ShapeDtypeStruct((B, 1, D), jnp.float32),
        input_output_aliases=aliases,
        compiler_params=pltpu.CompilerParams(
            dimension_semantics=("arbitrary",),
        ),
    )(*args)


def kernel(semantic_code, acoustic_codes, table):
    sem_idx = semantic_code.reshape(B).astype(jnp.int32) + 2
    codes = acoustic_codes.astype(jnp.int32)
    tac = jnp.concatenate(
        [
            table[AC_BASE : AC_BASE + AC_ROWS],
            jnp.zeros((AC_PAD - AC_ROWS, D), jnp.float32),
        ]
    ).astype(jnp.bfloat16)
    s_slices = [
        _sc_gather(
            sem_idx[k * _BS : (k + 1) * _BS].reshape(_NW, _NCH, _CH), table
        )
        for k in range(_NSLICE)
    ]
    out = None
    for k in range(_NSLICE):
        out = _tc_combine(
            codes[k * _BS : (k + 1) * _BS],
            s_slices[k],
            tac,
            k * (_BS // _TB),
            prev=out,
        )
    return out
